# baseline (speedup 1.0000x reference)
import jax
import jax.numpy as jnp
from jax import lax
from jax.experimental import pallas as pl
from jax.experimental.pallas import tpu as pltpu
from jax.experimental.pallas import tpu_sc as plsc

N = 10000
E = 160000
IN_DIM = 256
HID = 128
HEADS = 8
OUT = 128

N_PAD = 10240
E_PAD = 163840
CHUNK = N_PAD // 16
NODE_BLK = 512


def _tc1_body(x_ref, w1_ref, alr_ref, feat_ref, el_ref, er_ref):
    f = jnp.dot(x_ref[...], w1_ref[...], preferred_element_type=jnp.float32)
    t = jnp.dot(f, alr_ref[...], preferred_element_type=jnp.float32).T
    feat_ref[...] = f.reshape(NODE_BLK * HEADS, HID)
    el_ref[...] = t[:HEADS]
    er_ref[...] = t[HEADS:]


def _tc2_body(num_ref, den_ref, b1_ref, w2_ref, al2_ref, ar2_ref,
              feat2_ref, el2_ref, er2_ref):
    den = jnp.sum(den_ref[...], axis=1)
    den = jnp.where(den > 0, den, 1.0)
    h1 = num_ref[...] / den[:, :, None] + b1_ref[...][:, None, :]
    h1 = jnp.where(h1 > 0, h1, jnp.exp(jnp.minimum(h1, 0.0)) - 1.0)
    f2 = jnp.zeros((NODE_BLK, OUT), jnp.float32)
    for h in range(HEADS):
        f2 = f2 + jnp.dot(h1[h], w2_ref[h], preferred_element_type=jnp.float32)
    feat2_ref[...] = f2
    el2_ref[...] = jnp.sum(f2 * al2_ref[...], axis=1)
    er2_ref[...] = jnp.sum(f2 * ar2_ref[...], axis=1)


def _tc3_body(num_ref, den_ref, b2_ref, out_ref):
    den = jnp.sum(den_ref[...], axis=0)
    den = jnp.where(den > 0, den, 1.0)
    out_ref[...] = (num_ref[0] + num_ref[1]) / den[:, None] + b2_ref[...]


def _zero_fill(ref, nwords):
    z = jnp.zeros((16,), jnp.float32)

    def st(k, carry):
        ref[pl.ds(k * 16, 16)] = z
        return carry

    lax.fori_loop(0, nwords // 16, st, 0)


def _edge_pass(h_off, base_e, nb, batch, src_hbm, dst_hbm, feat_hbm,
               el_v, er_v, den_v, sd0, sd1, gidx0, gidx1, wv0, wv1,
               rows0, rows1, sg0, sg1, ss0, ss1, acc_sh, idx_mul):
    g_per_b = batch // 16
    npair = nb // 2

    def prep(b, sd, gidx, wv):
        off = base_e + b * batch
        pltpu.sync_copy(src_hbm.at[pl.ds(off, batch)], sd[0])
        pltpu.sync_copy(dst_hbm.at[pl.ds(off, batch)], sd[1])

        def grp(g, c2):
            sl = pl.ds(g * 16, 16)
            s16 = sd[0][sl]
            d16 = sd[1][sl]
            gidx[sl] = s16 * idx_mul + h_off
            e16 = plsc.load_gather(el_v, [s16]) + plsc.load_gather(er_v, [d16])
            e16 = jnp.where(e16 > 0, e16, 0.2 * e16)
            w16 = jnp.exp(e16)
            wv[sl] = w16
            plsc.addupdate_scatter(den_v, [d16], w16)
            return c2

        lax.fori_loop(0, g_per_b, grp, 0)

    def scale(rows, wv):
        @plsc.parallel_loop(0, g_per_b)
        def scl(g):
            w16 = wv[pl.ds(g * 16, 16)]
            for i in range(16):
                e_i = g * 16 + i
                w = w16[i]
                for j in range(HID // 16):
                    sl = pl.ds(j * 16, 16)
                    rows[e_i, sl] = rows[e_i, sl] * w

    def wait_gather(rows, sem):
        pltpu.make_async_copy(feat_hbm.at[pl.ds(0, batch)], rows, sem).wait()

    def wait_scatter(rows, sem):
        pltpu.make_async_copy(rows, acc_sh.at[pl.ds(0, batch)], sem).wait()

    prep(0, sd0, gidx0, wv0)
    pltpu.async_copy(feat_hbm.at[gidx0], rows0, sg0)
    prep(1, sd1, gidx1, wv1)
    pltpu.async_copy(feat_hbm.at[gidx1], rows1, sg1)

    def pair(k, carry):
        wait_gather(rows0, sg0)
        scale(rows0, wv0)
        pltpu.async_copy(rows0, acc_sh.at[sd0[1]], ss0, add=True)
        wait_scatter(rows0, ss0)

        @pl.when(k < npair - 1)
        def _():
            prep(2 * k + 2, sd0, gidx0, wv0)
            pltpu.async_copy(feat_hbm.at[gidx0], rows0, sg0)

        wait_gather(rows1, sg1)
        scale(rows1, wv1)
        pltpu.async_copy(rows1, acc_sh.at[sd1[1]], ss1, add=True)
        wait_scatter(rows1, ss1)

        @pl.when(k < npair - 1)
        def _():
            prep(2 * k + 3, sd1, gidx1, wv1)
            pltpu.async_copy(feat_hbm.at[gidx1], rows1, sg1)

        return carry

    lax.fori_loop(0, npair, pair, 0)


def _flush(s, den_v, acc_sh, num_hbm, num_row0, den_dst):
    pltpu.sync_copy(den_v, den_dst)
    pltpu.sync_copy(acc_sh.at[pl.ds(s * CHUNK, CHUNK)],
                    num_hbm.at[pl.ds(num_row0, CHUNK)])


BATCH = 64


def _zero_rows(rows):
    z = jnp.zeros((16,), jnp.float32)

    def st(k, c2):
        for j in range(HID // 16):
            rows[k, pl.ds(j * 16, 16)] = z
        return c2

    lax.fori_loop(0, BATCH, st, 0)


def _zero_acc_chunk(s, rows0, acc_sh):
    def zc(t, c2):
        pltpu.sync_copy(rows0, acc_sh.at[pl.ds(s * CHUNK + t * BATCH, BATCH)])
        return c2

    lax.fori_loop(0, CHUNK // BATCH, zc, 0)


def _sc1_body(src_hbm, dst_hbm, feat_hbm, el_hbm, er_hbm, num_hbm, den_hbm,
              el_v, er_v, den_v, srcb0, dstb0, srcb1, dstb1,
              gidx0, gidx1, wv0, wv1,
              rows0, rows1, acc_sh, sg0, sg1, ss0, ss1):
    c = lax.axis_index("c")
    s = lax.axis_index("s")
    sd0 = (srcb0, dstb0)
    sd1 = (srcb1, dstb1)
    e_per_tile = E_PAD // 16
    nb = e_per_tile // BATCH

    def do_head(i, carry):
        h = c * (HEADS // 2) + i
        _zero_rows(rows0)
        _zero_acc_chunk(s, rows0, acc_sh)
        _zero_fill(den_v, N_PAD)
        pltpu.sync_copy(el_hbm.at[h], el_v)
        pltpu.sync_copy(er_hbm.at[h], er_v)
        plsc.subcore_barrier()

        _edge_pass(h, s * e_per_tile, nb, BATCH, src_hbm, dst_hbm, feat_hbm,
                   el_v, er_v, den_v, sd0, sd1, gidx0, gidx1, wv0, wv1,
                   rows0, rows1, sg0, sg1, ss0, ss1, acc_sh, idx_mul=HEADS)
        plsc.subcore_barrier()
        _flush(s, den_v, acc_sh, num_hbm, h * N_PAD + s * CHUNK,
               den_hbm.at[h * 16 + s])
        plsc.subcore_barrier()
        return carry

    lax.fori_loop(0, HEADS // 2, do_head, 0)


def _sc2_body(src_hbm, dst_hbm, feat_hbm, el_hbm, er_hbm, num_hbm, den_hbm,
              el_v, er_v, den_v, srcb0, dstb0, srcb1, dstb1,
              gidx0, gidx1, wv0, wv1,
              rows0, rows1, acc_sh, sg0, sg1, ss0, ss1):
    c = lax.axis_index("c")
    s = lax.axis_index("s")
    sd0 = (srcb0, dstb0)
    sd1 = (srcb1, dstb1)
    e_per_tile = E_PAD // 32
    nb = e_per_tile // BATCH

    _zero_rows(rows0)
    _zero_acc_chunk(s, rows0, acc_sh)
    _zero_fill(den_v, N_PAD)
    pltpu.sync_copy(el_hbm.at[0], el_v)
    pltpu.sync_copy(er_hbm.at[0], er_v)
    plsc.subcore_barrier()

    worker = c * 16 + s
    _edge_pass(0, worker * e_per_tile, nb, BATCH, src_hbm, dst_hbm, feat_hbm,
               el_v, er_v, den_v, sd0, sd1, gidx0, gidx1, wv0, wv1,
               rows0, rows1, sg0, sg1, ss0, ss1, acc_sh, idx_mul=1)
    plsc.subcore_barrier()
    _flush(s, den_v, acc_sh, num_hbm, c * N_PAD + s * CHUNK,
           den_hbm.at[c * 16 + s])


def _make_sc_call(body, batch, num_rows, den_rows):
    mesh = plsc.VectorSubcoreMesh(core_axis_name="c", subcore_axis_name="s")
    return pl.kernel(
        body,
        compiler_params=pltpu.CompilerParams(needs_layout_passes=False),
        out_type=(
            jax.ShapeDtypeStruct((num_rows, HID), jnp.float32),
            jax.ShapeDtypeStruct((den_rows * 16, N_PAD), jnp.float32),
        ),
        mesh=mesh,
        scratch_types=[
            pltpu.VMEM((N_PAD,), jnp.float32),
            pltpu.VMEM((N_PAD,), jnp.float32),
            pltpu.VMEM((N_PAD,), jnp.float32),
            pltpu.VMEM((batch,), jnp.int32),
            pltpu.VMEM((batch,), jnp.int32),
            pltpu.VMEM((batch,), jnp.int32),
            pltpu.VMEM((batch,), jnp.int32),
            pltpu.VMEM((batch,), jnp.int32),
            pltpu.VMEM((batch,), jnp.int32),
            pltpu.VMEM((batch,), jnp.float32),
            pltpu.VMEM((batch,), jnp.float32),
            pltpu.VMEM((batch, HID), jnp.float32),
            pltpu.VMEM((batch, HID), jnp.float32),
            pltpu.VMEM_SHARED((N_PAD, HID), jnp.float32),
            pltpu.SemaphoreType.DMA,
            pltpu.SemaphoreType.DMA,
            pltpu.SemaphoreType.DMA,
            pltpu.SemaphoreType.DMA,
        ],
    )


def kernel(features, edge_index, W1, al1, ar1, b1, W2, al2, ar2, b2):
    f32 = jnp.float32
    xp = jnp.zeros((N_PAD, IN_DIM), f32).at[:N].set(features)
    pad_e = E_PAD - E
    srcp = jnp.concatenate([edge_index[0], jnp.zeros((pad_e,), jnp.int32)])
    dstp = jnp.concatenate([edge_index[1],
                            jnp.full((pad_e,), N_PAD - 1, jnp.int32)])

    eye = jnp.eye(HEADS, dtype=f32)
    alr1 = jnp.concatenate(
        [(al1[0][:, :, None] * eye[:, None, :]).reshape(HEADS * HID, HEADS),
         (ar1[0][:, :, None] * eye[:, None, :]).reshape(HEADS * HID, HEADS)],
        axis=1)

    grid = (N_PAD // NODE_BLK,)
    feat1, el1, er1 = pl.pallas_call(
        _tc1_body,
        grid=grid,
        in_specs=[
            pl.BlockSpec((NODE_BLK, IN_DIM), lambda i: (i, 0)),
            pl.BlockSpec((IN_DIM, HEADS * HID), lambda i: (0, 0)),
            pl.BlockSpec((HEADS * HID, 2 * HEADS), lambda i: (0, 0)),
        ],
        out_specs=[
            pl.BlockSpec((NODE_BLK * HEADS, HID), lambda i: (i, 0)),
            pl.BlockSpec((HEADS, NODE_BLK), lambda i: (0, i)),
            pl.BlockSpec((HEADS, NODE_BLK), lambda i: (0, i)),
        ],
        out_shape=[
            jax.ShapeDtypeStruct((N_PAD * HEADS, HID), f32),
            jax.ShapeDtypeStruct((HEADS, N_PAD), f32),
            jax.ShapeDtypeStruct((HEADS, N_PAD), f32),
        ],
    )(xp, W1, alr1)

    sc1 = _make_sc_call(_sc1_body, BATCH, HEADS * N_PAD, HEADS)
    num1, den1 = sc1(srcp, dstp, feat1, el1, er1)

    feat2, el2, er2 = pl.pallas_call(
        _tc2_body,
        grid=grid,
        in_specs=[
            pl.BlockSpec((HEADS, NODE_BLK, HID), lambda i: (0, i, 0)),
            pl.BlockSpec((HEADS, 16, NODE_BLK), lambda i: (0, 0, i)),
            pl.BlockSpec((HEADS, HID), lambda i: (0, 0)),
            pl.BlockSpec((HEADS, HID, OUT), lambda i: (0, 0, 0)),
            pl.BlockSpec((1, OUT), lambda i: (0, 0)),
            pl.BlockSpec((1, OUT), lambda i: (0, 0)),
        ],
        out_specs=[
            pl.BlockSpec((NODE_BLK, OUT), lambda i: (i, 0)),
            pl.BlockSpec((NODE_BLK,), lambda i: (i,)),
            pl.BlockSpec((NODE_BLK,), lambda i: (i,)),
        ],
        out_shape=[
            jax.ShapeDtypeStruct((N_PAD, OUT), f32),
            jax.ShapeDtypeStruct((N_PAD,), f32),
            jax.ShapeDtypeStruct((N_PAD,), f32),
        ],
    )(num1.reshape(HEADS, N_PAD, HID), den1.reshape(HEADS, 16, N_PAD),
      b1.reshape(HEADS, HID),
      W2.reshape(HEADS, HID, OUT), al2.reshape(1, OUT), ar2.reshape(1, OUT))

    sc2 = _make_sc_call(_sc2_body, BATCH, 2 * N_PAD, 2)
    num2, den2 = sc2(srcp, dstp, feat2, el2.reshape(1, N_PAD),
                     er2.reshape(1, N_PAD))

    out = pl.pallas_call(
        _tc3_body,
        grid=grid,
        in_specs=[
            pl.BlockSpec((2, NODE_BLK, OUT), lambda i: (0, i, 0)),
            pl.BlockSpec((32, NODE_BLK), lambda i: (0, i)),
            pl.BlockSpec((1, OUT), lambda i: (0, 0)),
        ],
        out_specs=pl.BlockSpec((NODE_BLK, OUT), lambda i: (i, 0)),
        out_shape=jax.ShapeDtypeStruct((N_PAD, OUT), f32),
    )(num2.reshape(2, N_PAD, OUT), den2, b2.reshape(1, OUT))

    return out[:N].reshape(N, 1, OUT)


# speedup vs baseline: 1.0668x; 1.0668x over previous
import jax
import jax.numpy as jnp
from jax import lax
from jax.experimental import pallas as pl
from jax.experimental.pallas import tpu as pltpu
from jax.experimental.pallas import tpu_sc as plsc

N = 10000
E = 160000
IN_DIM = 256
HID = 128
HEADS = 8
OUT = 128

N_PAD = 10240
E_PAD = 163840
CHUNK = N_PAD // 16
NODE_BLK = 512


def _tc1_body(x_ref, w1_ref, alr_ref, feat_ref, el_ref, er_ref):
    f = jnp.dot(x_ref[...], w1_ref[...], preferred_element_type=jnp.float32)
    t = jnp.dot(f, alr_ref[...], preferred_element_type=jnp.float32).T
    feat_ref[...] = f.reshape(NODE_BLK * HEADS, HID)
    el_ref[...] = t[:HEADS]
    er_ref[...] = t[HEADS:]


def _tc2_body(num_ref, den_ref, b1_ref, w2_ref, al2_ref, ar2_ref,
              feat2_ref, el2_ref, er2_ref):
    den = jnp.sum(den_ref[...], axis=1)
    den = jnp.where(den > 0, den, 1.0)
    h1 = num_ref[...] / den[:, :, None] + b1_ref[...][:, None, :]
    h1 = jnp.where(h1 > 0, h1, jnp.exp(jnp.minimum(h1, 0.0)) - 1.0)
    f2 = jnp.zeros((NODE_BLK, OUT), jnp.float32)
    for h in range(HEADS):
        f2 = f2 + jnp.dot(h1[h], w2_ref[h], preferred_element_type=jnp.float32)
    feat2_ref[...] = f2
    el2_ref[...] = jnp.sum(f2 * al2_ref[...], axis=1)
    er2_ref[...] = jnp.sum(f2 * ar2_ref[...], axis=1)


def _tc3_body(num_ref, den_ref, b2_ref, out_ref):
    den = jnp.sum(den_ref[...], axis=0)
    den = jnp.where(den > 0, den, 1.0)
    out_ref[...] = (num_ref[0] + num_ref[1]) / den[:, None] + b2_ref[...]


def _zero_fill(ref, nwords):
    z = jnp.zeros((16,), jnp.float32)

    def st(k, carry):
        ref[pl.ds(k * 16, 16)] = z
        return carry

    lax.fori_loop(0, nwords // 16, st, 0)


def _edge_pass(h_off, base_e, nb, batch, src_hbm, dst_hbm, feat_hbm,
               el_v, er_v, den_v, sd0, sd1, gidx0, gidx1, wv0, wv1,
               rows0, rows1, dsc0, dsc1, sg0, sg1, ss0, ss1, acc_sh,
               idx_mul):
    g_per_b = batch // 16
    npair = nb // 2

    def prep(b, sd, gidx, wv):
        off = base_e + b * batch
        pltpu.sync_copy(src_hbm.at[pl.ds(off, batch)], sd[0])
        pltpu.sync_copy(dst_hbm.at[pl.ds(off, batch)], sd[1])

        def grp(g, c2):
            sl = pl.ds(g * 16, 16)
            s16 = sd[0][sl]
            d16 = sd[1][sl]
            gidx[sl] = s16 * idx_mul + h_off
            e16 = plsc.load_gather(el_v, [s16]) + plsc.load_gather(er_v, [d16])
            e16 = jnp.where(e16 > 0, e16, 0.2 * e16)
            w16 = jnp.exp(e16)
            wv[sl] = w16
            plsc.addupdate_scatter(den_v, [d16], w16)
            return c2

        lax.fori_loop(0, g_per_b, grp, 0)

    def scale(rows, wv):
        @plsc.parallel_loop(0, g_per_b)
        def scl(g):
            w16 = wv[pl.ds(g * 16, 16)]
            for i in range(16):
                e_i = g * 16 + i
                w = w16[i]
                for j in range(HID // 16):
                    sl = pl.ds(j * 16, 16)
                    rows[e_i, sl] = rows[e_i, sl] * w

    def wait_gather(rows, sem):
        pltpu.make_async_copy(feat_hbm.at[pl.ds(0, batch)], rows, sem).wait()

    def wait_scatter(rows, sem):
        pltpu.make_async_copy(rows, acc_sh.at[pl.ds(0, batch)], sem).wait()

    prep(0, sd0, gidx0, wv0)
    pltpu.async_copy(feat_hbm.at[gidx0], rows0, sg0)
    prep(1, sd1, gidx1, wv1)
    pltpu.async_copy(feat_hbm.at[gidx1], rows1, sg1)

    def snap_dst(sd, dsc):
        for g in range(g_per_b):
            sl = pl.ds(g * 16, 16)
            dsc[sl] = sd[1][sl]

    def pair(k, carry):
        wait_gather(rows0, sg0)
        scale(rows0, wv0)
        snap_dst(sd0, dsc0)
        pltpu.async_copy(rows0, acc_sh.at[dsc0], ss0, add=True)

        @pl.when(k < npair - 1)
        def _():
            prep(2 * k + 2, sd0, gidx0, wv0)

        wait_scatter(rows0, ss0)

        @pl.when(k < npair - 1)
        def _():
            pltpu.async_copy(feat_hbm.at[gidx0], rows0, sg0)

        wait_gather(rows1, sg1)
        scale(rows1, wv1)
        snap_dst(sd1, dsc1)
        pltpu.async_copy(rows1, acc_sh.at[dsc1], ss1, add=True)

        @pl.when(k < npair - 1)
        def _():
            prep(2 * k + 3, sd1, gidx1, wv1)

        wait_scatter(rows1, ss1)

        @pl.when(k < npair - 1)
        def _():
            pltpu.async_copy(feat_hbm.at[gidx1], rows1, sg1)

        return carry

    lax.fori_loop(0, npair, pair, 0)


def _flush(s, den_v, acc_sh, num_hbm, num_row0, den_dst):
    pltpu.sync_copy(den_v, den_dst)
    pltpu.sync_copy(acc_sh.at[pl.ds(s * CHUNK, CHUNK)],
                    num_hbm.at[pl.ds(num_row0, CHUNK)])


BATCH = 64


def _zero_rows(rows):
    z = jnp.zeros((16,), jnp.float32)

    def st(k, c2):
        for j in range(HID // 16):
            rows[k, pl.ds(j * 16, 16)] = z
        return c2

    lax.fori_loop(0, BATCH, st, 0)


def _zero_acc_chunk(s, rows0, acc_sh):
    def zc(t, c2):
        pltpu.sync_copy(rows0, acc_sh.at[pl.ds(s * CHUNK + t * BATCH, BATCH)])
        return c2

    lax.fori_loop(0, CHUNK // BATCH, zc, 0)


def _sc1_body(src_hbm, dst_hbm, feat_hbm, el_hbm, er_hbm, num_hbm, den_hbm,
              el_v, er_v, den_v, srcb0, dstb0, srcb1, dstb1,
              gidx0, gidx1, wv0, wv1,
              rows0, rows1, dsc0, dsc1, acc_sh, sg0, sg1, ss0, ss1):
    c = lax.axis_index("c")
    s = lax.axis_index("s")
    sd0 = (srcb0, dstb0)
    sd1 = (srcb1, dstb1)
    e_per_tile = E_PAD // 16
    nb = e_per_tile // BATCH

    def do_head(i, carry):
        h = c * (HEADS // 2) + i
        _zero_rows(rows0)
        _zero_acc_chunk(s, rows0, acc_sh)
        _zero_fill(den_v, N_PAD)
        pltpu.sync_copy(el_hbm.at[h], el_v)
        pltpu.sync_copy(er_hbm.at[h], er_v)
        plsc.subcore_barrier()

        _edge_pass(h, s * e_per_tile, nb, BATCH, src_hbm, dst_hbm, feat_hbm,
                   el_v, er_v, den_v, sd0, sd1, gidx0, gidx1, wv0, wv1,
                   rows0, rows1, dsc0, dsc1, sg0, sg1, ss0, ss1, acc_sh,
                   idx_mul=HEADS)
        plsc.subcore_barrier()
        _flush(s, den_v, acc_sh, num_hbm, h * N_PAD + s * CHUNK,
               den_hbm.at[h * 16 + s])
        plsc.subcore_barrier()
        return carry

    lax.fori_loop(0, HEADS // 2, do_head, 0)


def _sc2_body(src_hbm, dst_hbm, feat_hbm, el_hbm, er_hbm, num_hbm, den_hbm,
              el_v, er_v, den_v, srcb0, dstb0, srcb1, dstb1,
              gidx0, gidx1, wv0, wv1,
              rows0, rows1, dsc0, dsc1, acc_sh, sg0, sg1, ss0, ss1):
    c = lax.axis_index("c")
    s = lax.axis_index("s")
    sd0 = (srcb0, dstb0)
    sd1 = (srcb1, dstb1)
    e_per_tile = E_PAD // 32
    nb = e_per_tile // BATCH

    _zero_rows(rows0)
    _zero_acc_chunk(s, rows0, acc_sh)
    _zero_fill(den_v, N_PAD)
    pltpu.sync_copy(el_hbm.at[0], el_v)
    pltpu.sync_copy(er_hbm.at[0], er_v)
    plsc.subcore_barrier()

    worker = c * 16 + s
    _edge_pass(0, worker * e_per_tile, nb, BATCH, src_hbm, dst_hbm, feat_hbm,
               el_v, er_v, den_v, sd0, sd1, gidx0, gidx1, wv0, wv1,
               rows0, rows1, dsc0, dsc1, sg0, sg1, ss0, ss1, acc_sh,
               idx_mul=1)
    plsc.subcore_barrier()
    _flush(s, den_v, acc_sh, num_hbm, c * N_PAD + s * CHUNK,
           den_hbm.at[c * 16 + s])


def _make_sc_call(body, batch, num_rows, den_rows):
    mesh = plsc.VectorSubcoreMesh(core_axis_name="c", subcore_axis_name="s")
    return pl.kernel(
        body,
        compiler_params=pltpu.CompilerParams(needs_layout_passes=False),
        out_type=(
            jax.ShapeDtypeStruct((num_rows, HID), jnp.float32),
            jax.ShapeDtypeStruct((den_rows * 16, N_PAD), jnp.float32),
        ),
        mesh=mesh,
        scratch_types=[
            pltpu.VMEM((N_PAD,), jnp.float32),
            pltpu.VMEM((N_PAD,), jnp.float32),
            pltpu.VMEM((N_PAD,), jnp.float32),
            pltpu.VMEM((batch,), jnp.int32),
            pltpu.VMEM((batch,), jnp.int32),
            pltpu.VMEM((batch,), jnp.int32),
            pltpu.VMEM((batch,), jnp.int32),
            pltpu.VMEM((batch,), jnp.int32),
            pltpu.VMEM((batch,), jnp.int32),
            pltpu.VMEM((batch,), jnp.float32),
            pltpu.VMEM((batch,), jnp.float32),
            pltpu.VMEM((batch, HID), jnp.float32),
            pltpu.VMEM((batch, HID), jnp.float32),
            pltpu.VMEM((batch,), jnp.int32),
            pltpu.VMEM((batch,), jnp.int32),
            pltpu.VMEM_SHARED((N_PAD, HID), jnp.float32),
            pltpu.SemaphoreType.DMA,
            pltpu.SemaphoreType.DMA,
            pltpu.SemaphoreType.DMA,
            pltpu.SemaphoreType.DMA,
        ],
    )


def kernel(features, edge_index, W1, al1, ar1, b1, W2, al2, ar2, b2):
    f32 = jnp.float32
    xp = jnp.zeros((N_PAD, IN_DIM), f32).at[:N].set(features)
    pad_e = E_PAD - E
    srcp = jnp.concatenate([edge_index[0], jnp.zeros((pad_e,), jnp.int32)])
    dstp = jnp.concatenate([edge_index[1],
                            jnp.full((pad_e,), N_PAD - 1, jnp.int32)])

    eye = jnp.eye(HEADS, dtype=f32)
    alr1 = jnp.concatenate(
        [(al1[0][:, :, None] * eye[:, None, :]).reshape(HEADS * HID, HEADS),
         (ar1[0][:, :, None] * eye[:, None, :]).reshape(HEADS * HID, HEADS)],
        axis=1)

    grid = (N_PAD // NODE_BLK,)
    feat1, el1, er1 = pl.pallas_call(
        _tc1_body,
        grid=grid,
        in_specs=[
            pl.BlockSpec((NODE_BLK, IN_DIM), lambda i: (i, 0)),
            pl.BlockSpec((IN_DIM, HEADS * HID), lambda i: (0, 0)),
            pl.BlockSpec((HEADS * HID, 2 * HEADS), lambda i: (0, 0)),
        ],
        out_specs=[
            pl.BlockSpec((NODE_BLK * HEADS, HID), lambda i: (i, 0)),
            pl.BlockSpec((HEADS, NODE_BLK), lambda i: (0, i)),
            pl.BlockSpec((HEADS, NODE_BLK), lambda i: (0, i)),
        ],
        out_shape=[
            jax.ShapeDtypeStruct((N_PAD * HEADS, HID), f32),
            jax.ShapeDtypeStruct((HEADS, N_PAD), f32),
            jax.ShapeDtypeStruct((HEADS, N_PAD), f32),
        ],
    )(xp, W1, alr1)

    sc1 = _make_sc_call(_sc1_body, BATCH, HEADS * N_PAD, HEADS)
    num1, den1 = sc1(srcp, dstp, feat1, el1, er1)

    feat2, el2, er2 = pl.pallas_call(
        _tc2_body,
        grid=grid,
        in_specs=[
            pl.BlockSpec((HEADS, NODE_BLK, HID), lambda i: (0, i, 0)),
            pl.BlockSpec((HEADS, 16, NODE_BLK), lambda i: (0, 0, i)),
            pl.BlockSpec((HEADS, HID), lambda i: (0, 0)),
            pl.BlockSpec((HEADS, HID, OUT), lambda i: (0, 0, 0)),
            pl.BlockSpec((1, OUT), lambda i: (0, 0)),
            pl.BlockSpec((1, OUT), lambda i: (0, 0)),
        ],
        out_specs=[
            pl.BlockSpec((NODE_BLK, OUT), lambda i: (i, 0)),
            pl.BlockSpec((NODE_BLK,), lambda i: (i,)),
            pl.BlockSpec((NODE_BLK,), lambda i: (i,)),
        ],
        out_shape=[
            jax.ShapeDtypeStruct((N_PAD, OUT), f32),
            jax.ShapeDtypeStruct((N_PAD,), f32),
            jax.ShapeDtypeStruct((N_PAD,), f32),
        ],
    )(num1.reshape(HEADS, N_PAD, HID), den1.reshape(HEADS, 16, N_PAD),
      b1.reshape(HEADS, HID),
      W2.reshape(HEADS, HID, OUT), al2.reshape(1, OUT), ar2.reshape(1, OUT))

    sc2 = _make_sc_call(_sc2_body, BATCH, 2 * N_PAD, 2)
    num2, den2 = sc2(srcp, dstp, feat2, el2.reshape(1, N_PAD),
                     er2.reshape(1, N_PAD))

    out = pl.pallas_call(
        _tc3_body,
        grid=grid,
        in_specs=[
            pl.BlockSpec((2, NODE_BLK, OUT), lambda i: (0, i, 0)),
            pl.BlockSpec((32, NODE_BLK), lambda i: (0, i)),
            pl.BlockSpec((1, OUT), lambda i: (0, 0)),
        ],
        out_specs=pl.BlockSpec((NODE_BLK, OUT), lambda i: (i, 0)),
        out_shape=jax.ShapeDtypeStruct((N_PAD, OUT), f32),
    )(num2.reshape(2, N_PAD, OUT), den2, b2.reshape(1, OUT))

    return out[:N].reshape(N, 1, OUT)
